# baseline (speedup 1.0000x reference)
import jax
import jax.numpy as jnp
from jax import lax
from jax.experimental import pallas as pl
from jax.experimental.pallas import tpu as pltpu
from jax.experimental.pallas import tpu_sc as plsc

_INFO = plsc.get_sparse_core_info()
_NC = _INFO.num_cores
_NS = _INFO.num_subcores
_NW = _NC * _NS
_L = _INFO.num_lanes

_B = 1024 * 20
_D = 2048
_V = 38
_BPW = _B // _NW
_CH = 8


def _body(table_hbm, idx_hbm, out_hbm, table_v, idx_v, buf, sem0, sem1):
    wid = lax.axis_index("s") * _NC + lax.axis_index("c")
    base = wid * _BPW
    pltpu.sync_copy(idx_hbm.at[pl.ds(base, _BPW)], idx_v)
    pltpu.sync_copy(table_hbm, table_v)

    sems = (sem0, sem1)

    def stream_wait(p):
        pltpu.make_async_copy(
            buf.at[p], out_hbm.at[pl.ds(base, _CH)], sems[p]
        ).wait()

    @pl.loop(0, _BPW, step=2 * _CH)
    def _(g):
        vals = idx_v[pl.ds(g, _L)]
        for p in range(2):
            row0 = g + p * _CH

            @pl.when(g > 0)
            def _():
                stream_wait(p)

            vs = [vals[p * _CH + r] for r in range(_CH)]

            @pl.loop(0, _D // _L, step=4)
            def _(j):
                for jj in range(4):
                    col = (j + jj) * _L
                    for r in range(_CH):
                        buf[p, r, pl.ds(col, _L)] = table_v[vs[r], pl.ds(col, _L)]

            pltpu.async_copy(
                buf.at[p], out_hbm.at[pl.ds(base + row0, _CH)], sems[p]
            )

    for p in range(2):
        stream_wait(p)


def kernel(signal, table):
    idx = signal.reshape(-1).astype(jnp.int32)
    mesh = plsc.VectorSubcoreMesh(core_axis_name="c", subcore_axis_name="s")
    run = pl.kernel(
        _body,
        mesh=mesh,
        out_type=jax.ShapeDtypeStruct((_B, _D), jnp.float32),
        scratch_types=[
            pltpu.VMEM((_V, _D), jnp.float32),
            pltpu.VMEM((_BPW,), jnp.int32),
            pltpu.VMEM((2, _CH, _D), jnp.float32),
            pltpu.SemaphoreType.DMA,
            pltpu.SemaphoreType.DMA,
        ],
    )
    out = run(table, idx)
    return out.reshape(signal.shape + (_D,))


# speedup vs baseline: 1.6079x; 1.6079x over previous
import jax
import jax.numpy as jnp
from jax import lax
from jax.experimental import pallas as pl
from jax.experimental.pallas import tpu as pltpu
from jax.experimental.pallas import tpu_sc as plsc

_INFO = plsc.get_sparse_core_info()
_NC = _INFO.num_cores
_NS = _INFO.num_subcores
_NW = _NC * _NS

_B = 1024 * 20
_D = 2048
_V = 38
_BPW = _B // _NW
_K = 16


def _body(table_hbm, idx_hbm, out_hbm, table_v, idx_v, *sems):
    wid = lax.axis_index("s") * _NC + lax.axis_index("c")
    base = wid * _BPW
    pltpu.sync_copy(idx_hbm.at[pl.ds(base, _BPW)], idx_v)
    pltpu.sync_copy(table_hbm, table_v)

    def row_start(i, v, b):
        pltpu.async_copy(table_v.at[v], out_hbm.at[base + i], sems[b])

    def row_wait(b):
        pltpu.make_async_copy(table_v.at[0], out_hbm.at[base], sems[b]).wait()

    @pl.loop(0, _BPW, step=_K)
    def _(g):
        @pl.when(g > 0)
        def _():
            for b in range(_K):
                row_wait(b)

        vals = idx_v[pl.ds(g, _K)]
        for b in range(_K):
            row_start(g + b, vals[b], b)

    for b in range(_K):
        row_wait(b)


def kernel(signal, table):
    idx = signal.reshape(-1).astype(jnp.int32)
    mesh = plsc.VectorSubcoreMesh(core_axis_name="c", subcore_axis_name="s")
    run = pl.kernel(
        _body,
        mesh=mesh,
        out_type=jax.ShapeDtypeStruct((_B, _D), jnp.float32),
        scratch_types=[
            pltpu.VMEM((_V, _D), jnp.float32),
            pltpu.VMEM((_BPW,), jnp.int32),
        ]
        + [pltpu.SemaphoreType.DMA] * _K,
    )
    out = run(table, idx)
    return out.reshape(signal.shape + (_D,))


# speedup vs baseline: 2.7667x; 1.7207x over previous
import jax
import jax.numpy as jnp
from jax import lax
from jax.experimental import pallas as pl
from jax.experimental.pallas import tpu as pltpu
from jax.experimental.pallas import tpu_sc as plsc

_INFO = plsc.get_sparse_core_info()
_NC = _INFO.num_cores
_NS = _INFO.num_subcores
_NW = _NC * _NS

_N = 1024
_S = 20
_B = _N * _S
_D = 2048
_V = 38
_BPW = _B // _NW
_K = 16


def _body(table_hbm, idx_hbm, out_hbm, table_v, idx_v, *sems):
    wid = lax.axis_index("s") * _NC + lax.axis_index("c")
    base = wid * _BPW
    pltpu.sync_copy(idx_hbm.at[pl.ds(base, _BPW)], idx_v)
    pltpu.sync_copy(table_hbm, table_v)

    def row_start(q, r, v, b):
        pltpu.async_copy(table_v.at[v], out_hbm.at[q, r], sems[b])

    def row_wait(b):
        pltpu.make_async_copy(table_v.at[0], out_hbm.at[0, 0], sems[b]).wait()

    @pl.loop(0, _BPW, step=_K)
    def _(g):
        @pl.when(g > 0)
        def _():
            for b in range(_K):
                row_wait(b)

        vals = idx_v[pl.ds(g, _K)]
        f0 = base + g
        q = f0 // _S
        r = f0 - q * _S
        for b in range(_K):
            row_start(q, r, vals[b], b)
            is_last = r == (_S - 1)
            q = q + jnp.where(is_last, 1, 0)
            r = jnp.where(is_last, 0, r + 1)

    for b in range(_K):
        row_wait(b)


def kernel(signal, table):
    idx = signal.reshape(-1).astype(jnp.int32)
    mesh = plsc.VectorSubcoreMesh(core_axis_name="c", subcore_axis_name="s")
    run = pl.kernel(
        _body,
        mesh=mesh,
        out_type=jax.ShapeDtypeStruct((_N, _S, _D), jnp.float32),
        scratch_types=[
            pltpu.VMEM((_V, _D), jnp.float32),
            pltpu.VMEM((_BPW,), jnp.int32),
        ]
        + [pltpu.SemaphoreType.DMA] * _K,
    )
    return run(table, idx)


# speedup vs baseline: 2.7778x; 1.0040x over previous
import jax
import jax.numpy as jnp
from jax import lax
from jax.experimental import pallas as pl
from jax.experimental.pallas import tpu as pltpu
from jax.experimental.pallas import tpu_sc as plsc

_INFO = plsc.get_sparse_core_info()
_NC = _INFO.num_cores
_NS = _INFO.num_subcores
_NW = _NC * _NS

_N = 1024
_S = 20
_B = _N * _S
_D = 2048
_V = 38
_BPW = _B // _NW
_K = 16


def _body(table_hbm, idx_hbm, out_hbm, table_v, idx_v, *sems):
    wid = lax.axis_index("s") * _NC + lax.axis_index("c")
    base = wid * _BPW
    pltpu.sync_copy(idx_hbm.at[pl.ds(base, _BPW)], idx_v)
    pltpu.sync_copy(table_hbm, table_v)

    def row_start(q, r, v, b):
        pltpu.async_copy(table_v.at[v], out_hbm.at[q, r], sems[b])

    def row_wait(b):
        pltpu.make_async_copy(table_v.at[0], out_hbm.at[0, 0], sems[b]).wait()

    @pl.loop(0, _BPW, step=_K)
    def _(g):
        @pl.when(g > 0)
        def _():
            for b in range(_K):
                row_wait(b)

        vals = idx_v[pl.ds(g, _K)]
        f0 = base + g
        q = f0 // _S
        r = f0 - q * _S
        for b in range(_K):
            row_start(q, r, vals[b], b)
            is_last = r == (_S - 1)
            q = q + jnp.where(is_last, 1, 0)
            r = jnp.where(is_last, 0, r + 1)

    for b in range(_K):
        row_wait(b)


def kernel(signal, table):
    idx = signal.reshape(-1).astype(jnp.int32)
    mesh = plsc.VectorSubcoreMesh(core_axis_name="c", subcore_axis_name="s")
    run = pl.kernel(
        _body,
        mesh=mesh,
        out_type=jax.ShapeDtypeStruct((_N, _S, _D), jnp.float32),
        compiler_params=pltpu.CompilerParams(use_tc_tiling_on_sc=True),
        scratch_types=[
            pltpu.VMEM((_V, _D), jnp.float32),
            pltpu.VMEM((_BPW,), jnp.int32),
        ]
        + [pltpu.SemaphoreType.DMA] * _K,
    )
    return run(table, idx)
